# baseline (speedup 1.0000x reference)
import functools

import jax
import jax.numpy as jnp
from jax import lax
from jax.experimental import pallas as pl
from jax.experimental.pallas import tpu as pltpu
from jax.experimental.pallas import tpu_sc as plsc

B = 4
S = 8192
D = 768
LANES = 16
NVEC = D // LANES

NC = 2
NS = 16
NW = NC * NS
S_PER_W = S // NW
C = 32
NCHUNK = S_PER_W // C

_mesh = plsc.VectorSubcoreMesh(core_axis_name="c", subcore_axis_name="s")


@functools.partial(
    pl.kernel,
    mesh=_mesh,
    out_type=jax.ShapeDtypeStruct((B, S, D), jnp.float32),
    scratch_types=[
        pltpu.VMEM((C,), jnp.int32),
        pltpu.VMEM((C, D), jnp.float32),
        pltpu.VMEM((C, D), jnp.float32),
        pltpu.SemaphoreType.DMA,
    ],
)
def _emb_lookup(x_hbm, emb_hbm, pos_hbm, out_hbm, idx_v, pos_v, emb_v, sem):
    wid = lax.axis_index("s") * NC + lax.axis_index("c")
    s_base = wid * S_PER_W

    def chunk_body(pc, _):
        s0 = s_base + pc * C
        pltpu.sync_copy(pos_hbm.at[pl.ds(s0, C), :], pos_v)

        def batch_body(b, _):
            pltpu.sync_copy(x_hbm.at[b, pl.ds(s0, C)], idx_v)
            pltpu.async_copy(emb_hbm.at[idx_v], emb_v, sem).wait()

            def row_body(r, _):
                def vec_body(j, _):
                    off = j * LANES
                    emb_v[r, pl.ds(off, LANES)] = (
                        emb_v[r, pl.ds(off, LANES)] + pos_v[r, pl.ds(off, LANES)]
                    )
                    return 0

                return lax.fori_loop(0, NVEC, vec_body, 0)

            lax.fori_loop(0, C, row_body, 0)
            pltpu.sync_copy(emb_v, out_hbm.at[b, pl.ds(s0, C), :])
            return 0

        lax.fori_loop(0, B, batch_body, 0)
        return 0

    lax.fori_loop(0, NCHUNK, chunk_body, 0)


def kernel(x, emb_table, pos_table):
    return _emb_lookup(x.astype(jnp.int32), emb_table, pos_table)


# speedup vs baseline: 1.4173x; 1.4173x over previous
import functools

import jax
import jax.numpy as jnp
from jax import lax
from jax.experimental import pallas as pl
from jax.experimental.pallas import tpu as pltpu
from jax.experimental.pallas import tpu_sc as plsc

B = 4
S = 8192
D = 768
LANES = 16
NVEC = D // LANES

NC = 2
NS = 16
NW = NC * NS
S_PER_W = S // NW
C = 32
NCHUNK = S_PER_W // C
NITEM = NCHUNK * B

_mesh = plsc.VectorSubcoreMesh(core_axis_name="c", subcore_axis_name="s")


@functools.partial(
    pl.kernel,
    mesh=_mesh,
    out_type=jax.ShapeDtypeStruct((B, S, D), jnp.float32),
    scratch_types=[
        pltpu.VMEM((B, S_PER_W), jnp.int32),
        pltpu.VMEM((3, C, D), jnp.float32),
        pltpu.VMEM((2, C, D), jnp.float32),
        pltpu.SemaphoreType.DMA,
        pltpu.SemaphoreType.DMA,
        pltpu.SemaphoreType.DMA,
        pltpu.SemaphoreType.DMA,
        pltpu.SemaphoreType.DMA,
        pltpu.SemaphoreType.DMA,
        pltpu.SemaphoreType.DMA,
        pltpu.SemaphoreType.DMA,
    ],
)
def _emb_lookup(x_hbm, emb_hbm, pos_hbm, out_hbm, idx_all, emb_v, pos_v,
                g0, g1, g2, o0, o1, o2, p0, p1):
    wid = lax.axis_index("s") * NC + lax.axis_index("c")
    s_base = wid * S_PER_W
    gsems = (g0, g1, g2)
    osems = (o0, o1, o2)
    psems = (p0, p1)

    for b in range(B):
        pltpu.sync_copy(x_hbm.at[b, pl.ds(s_base, S_PER_W)], idx_all.at[b])

    def start_pos(q):
        for k in range(2):
            @pl.when(q % 2 == k)
            def _():
                pltpu.async_copy(
                    pos_hbm.at[pl.ds(s_base + q * C, C), :], pos_v.at[k], psems[k])

    def wait_pos(q):
        for k in range(2):
            @pl.when(q % 2 == k)
            def _():
                pltpu.make_async_copy(
                    pos_hbm.at[pl.ds(s_base + q * C, C), :], pos_v.at[k], psems[k]
                ).wait()

    def _item_refs(t):
        b = lax.rem(t, B)
        pc = lax.div(t, B)
        s0 = s_base + pc * C
        idx_sl = idx_all.at[b, pl.ds(pc * C, C)]
        return b, s0, idx_sl

    def start_gather(t):
        _, _, idx_sl = _item_refs(t)
        for k in range(3):
            @pl.when(t % 3 == k)
            def _():
                pltpu.async_copy(emb_hbm.at[idx_sl], emb_v.at[k], gsems[k])

    def wait_gather(t):
        _, _, idx_sl = _item_refs(t)
        for k in range(3):
            @pl.when(t % 3 == k)
            def _():
                pltpu.make_async_copy(emb_hbm.at[idx_sl], emb_v.at[k], gsems[k]).wait()

    def start_out(t):
        b, s0, _ = _item_refs(t)
        for k in range(3):
            @pl.when(t % 3 == k)
            def _():
                pltpu.async_copy(emb_v.at[k], out_hbm.at[b, pl.ds(s0, C), :], osems[k])

    def wait_out(t):
        b, s0, _ = _item_refs(t)
        for k in range(3):
            @pl.when(t % 3 == k)
            def _():
                pltpu.make_async_copy(
                    emb_v.at[k], out_hbm.at[b, pl.ds(s0, C), :], osems[k]).wait()

    def add_item(t):
        par = lax.rem(t, 3)
        ppar = lax.rem(lax.div(t, B), 2)

        def row_body(r, _):
            for j in range(NVEC):
                off = j * LANES
                emb_v[par, r, pl.ds(off, LANES)] = (
                    emb_v[par, r, pl.ds(off, LANES)]
                    + pos_v[ppar, r, pl.ds(off, LANES)]
                )
            return 0

        lax.fori_loop(0, C, row_body, 0)

    start_pos(0)
    start_gather(0)

    def pipe_body(i, _):
        t_prev = i - 1

        @pl.when(i < NITEM)
        def _():
            @pl.when(lax.rem(i, B) == 0)
            def _():
                start_pos(lax.div(i, B))

            @pl.when(i >= 3)
            def _():
                wait_out(i - 3)

            start_gather(i)

        wait_gather(t_prev)

        @pl.when(lax.rem(t_prev, B) == 0)
        def _():
            wait_pos(lax.div(t_prev, B))

        add_item(t_prev)
        start_out(t_prev)
        return 0

    lax.fori_loop(1, NITEM + 1, pipe_body, 0)

    for t in range(NITEM - 3, NITEM):
        wait_out(t)


def kernel(x, emb_table, pos_table):
    return _emb_lookup(x.astype(jnp.int32), emb_table, pos_table)


# speedup vs baseline: 3.3366x; 2.3542x over previous
import functools

import jax
import jax.numpy as jnp
from jax import lax
from jax.experimental import pallas as pl
from jax.experimental.pallas import tpu as pltpu
from jax.experimental.pallas import tpu_sc as plsc

B = 4
S = 8192
D = 768
LANES = 16
NVEC = D // LANES

NC = 2
NS = 16
NW = NC * NS
S_PER_W = S // NW
C = 32
NCHUNK = S_PER_W // C
NITEM = NCHUNK * B

_mesh = plsc.VectorSubcoreMesh(core_axis_name="c", subcore_axis_name="s")


@functools.partial(
    pl.kernel,
    mesh=_mesh,
    out_type=jax.ShapeDtypeStruct((B, S, D), jnp.float32),
    scratch_types=[
        pltpu.VMEM((B, S_PER_W), jnp.int32),
        pltpu.VMEM((3, C, D), jnp.float32),
        pltpu.VMEM((2, C, D), jnp.float32),
        pltpu.SemaphoreType.DMA,
        pltpu.SemaphoreType.DMA,
        pltpu.SemaphoreType.DMA,
        pltpu.SemaphoreType.DMA,
        pltpu.SemaphoreType.DMA,
        pltpu.SemaphoreType.DMA,
        pltpu.SemaphoreType.DMA,
        pltpu.SemaphoreType.DMA,
    ],
)
def _emb_lookup(x_hbm, emb_hbm, pos_hbm, out_hbm, idx_all, emb_v, pos_v,
                g0, g1, g2, o0, o1, o2, p0, p1):
    wid = lax.axis_index("s") * NC + lax.axis_index("c")
    s_base = wid * S_PER_W
    gsems = (g0, g1, g2)
    osems = (o0, o1, o2)
    psems = (p0, p1)

    for b in range(B):
        pltpu.sync_copy(x_hbm.at[b, pl.ds(s_base, S_PER_W)], idx_all.at[b])

    def start_pos(q):
        for k in range(2):
            @pl.when(q % 2 == k)
            def _():
                pltpu.async_copy(
                    pos_hbm.at[pl.ds(s_base + q * C, C), :], pos_v.at[k], psems[k])

    def wait_pos(q):
        for k in range(2):
            @pl.when(q % 2 == k)
            def _():
                pltpu.make_async_copy(
                    pos_hbm.at[pl.ds(s_base + q * C, C), :], pos_v.at[k], psems[k]
                ).wait()

    def _item_refs(t):
        b = lax.rem(t, B)
        pc = lax.div(t, B)
        s0 = s_base + pc * C
        idx_sl = idx_all.at[b, pl.ds(pc * C, C)]
        return b, s0, idx_sl

    def start_gather(t):
        _, _, idx_sl = _item_refs(t)
        for k in range(3):
            @pl.when(t % 3 == k)
            def _():
                pltpu.async_copy(emb_hbm.at[idx_sl], emb_v.at[k], gsems[k])

    def wait_gather(t):
        _, _, idx_sl = _item_refs(t)
        for k in range(3):
            @pl.when(t % 3 == k)
            def _():
                pltpu.make_async_copy(emb_hbm.at[idx_sl], emb_v.at[k], gsems[k]).wait()

    def start_out(t):
        b, s0, _ = _item_refs(t)
        for k in range(3):
            @pl.when(t % 3 == k)
            def _():
                pltpu.async_copy(emb_v.at[k], out_hbm.at[b, pl.ds(s0, C), :], osems[k])

    def wait_out(t):
        b, s0, _ = _item_refs(t)
        for k in range(3):
            @pl.when(t % 3 == k)
            def _():
                pltpu.make_async_copy(
                    emb_v.at[k], out_hbm.at[b, pl.ds(s0, C), :], osems[k]).wait()

    def add_item(t):
        par = lax.rem(t, 3)
        ppar = lax.rem(lax.div(t, B), 2)
        BLK = 8

        def row_body(r, _):
            for jb in range(NVEC // BLK):
                base = jb * BLK * LANES
                es = [emb_v[par, r, pl.ds(base + j * LANES, LANES)]
                      for j in range(BLK)]
                ps = [pos_v[ppar, r, pl.ds(base + j * LANES, LANES)]
                      for j in range(BLK)]
                ss = [e + p for e, p in zip(es, ps)]
                for j in range(BLK):
                    emb_v[par, r, pl.ds(base + j * LANES, LANES)] = ss[j]
            return 0

        lax.fori_loop(0, C, row_body, 0)

    start_pos(0)
    start_gather(0)

    def pipe_body(i, _):
        t_prev = i - 1

        @pl.when(i < NITEM)
        def _():
            @pl.when(lax.rem(i, B) == 0)
            def _():
                start_pos(lax.div(i, B))

            @pl.when(i >= 3)
            def _():
                wait_out(i - 3)

            start_gather(i)

        wait_gather(t_prev)

        @pl.when(lax.rem(t_prev, B) == 0)
        def _():
            wait_pos(lax.div(t_prev, B))

        add_item(t_prev)
        start_out(t_prev)
        return 0

    lax.fori_loop(1, NITEM + 1, pipe_body, 0)

    for t in range(NITEM - 3, NITEM):
        wait_out(t)


def kernel(x, emb_table, pos_table):
    return _emb_lookup(x.astype(jnp.int32), emb_table, pos_table)


# speedup vs baseline: 3.4500x; 1.0340x over previous
import functools

import jax
import jax.numpy as jnp
from jax import lax
from jax.experimental import pallas as pl
from jax.experimental.pallas import tpu as pltpu
from jax.experimental.pallas import tpu_sc as plsc

B = 4
S = 8192
D = 768
LANES = 16
NVEC = D // LANES
BLK = 8
NBLK = NVEC // BLK

NC = 2
NS = 16
NW = NC * NS
S_PER_W = S // NW
C = 8
NITEM = S_PER_W // C
NBUF = 4

_mesh = plsc.VectorSubcoreMesh(core_axis_name="c", subcore_axis_name="s")


@functools.partial(
    pl.kernel,
    mesh=_mesh,
    out_type=jax.ShapeDtypeStruct((B, S, D), jnp.float32),
    scratch_types=[
        pltpu.VMEM((B, S_PER_W), jnp.int32),
        pltpu.VMEM((NBUF, B, C, D), jnp.float32),
        pltpu.VMEM((NBUF, C, D), jnp.float32),
    ] + [pltpu.SemaphoreType.DMA] * (3 * NBUF),
)
def _emb_lookup(x_hbm, emb_hbm, pos_hbm, out_hbm, idx_all, embs, pos_v, *sems):
    gsems = sems[0:NBUF]
    osems = sems[NBUF:2 * NBUF]
    psems = sems[2 * NBUF:3 * NBUF]
    wid = lax.axis_index("s") * NC + lax.axis_index("c")
    s_base = wid * S_PER_W

    for b in range(B):
        pltpu.sync_copy(x_hbm.at[b, pl.ds(s_base, S_PER_W)], idx_all.at[b])

    def start_item(t):
        for k in range(NBUF):
            @pl.when(lax.rem(t, NBUF) == k)
            def _():
                pltpu.async_copy(
                    pos_hbm.at[pl.ds(s_base + t * C, C), :], pos_v.at[k], psems[k])
                for b in range(B):
                    pltpu.async_copy(
                        emb_hbm.at[idx_all.at[b, pl.ds(t * C, C)]],
                        embs.at[k, b], gsems[k])

    def wait_item(t):
        for k in range(NBUF):
            @pl.when(lax.rem(t, NBUF) == k)
            def _():
                pltpu.make_async_copy(
                    pos_hbm.at[pl.ds(s_base + t * C, C), :], pos_v.at[k], psems[k]
                ).wait()
                for b in range(B):
                    pltpu.make_async_copy(
                        emb_hbm.at[idx_all.at[b, pl.ds(t * C, C)]],
                        embs.at[k, b], gsems[k]).wait()

    def start_out(t):
        s0 = s_base + t * C
        for k in range(NBUF):
            @pl.when(lax.rem(t, NBUF) == k)
            def _():
                for b in range(B):
                    pltpu.async_copy(
                        embs.at[k, b], out_hbm.at[b, pl.ds(s0, C), :], osems[k])

    def wait_out(t):
        s0 = s_base + t * C
        for k in range(NBUF):
            @pl.when(lax.rem(t, NBUF) == k)
            def _():
                for b in range(B):
                    pltpu.make_async_copy(
                        embs.at[k, b], out_hbm.at[b, pl.ds(s0, C), :], osems[k]
                    ).wait()

    def add_item(t):
        par = lax.rem(t, NBUF)

        def row_body(r, _):
            for jb in range(NBLK):
                base = jb * BLK * LANES
                ps = [pos_v[par, r, pl.ds(base + j * LANES, LANES)]
                      for j in range(BLK)]
                for b in range(B):
                    es = [embs[par, b, r, pl.ds(base + j * LANES, LANES)]
                          for j in range(BLK)]
                    ss = [e + p for e, p in zip(es, ps)]
                    for j in range(BLK):
                        embs[par, b, r, pl.ds(base + j * LANES, LANES)] = ss[j]
            return 0

        lax.fori_loop(0, C, row_body, 0)

    start_item(0)
    start_item(1)

    def pipe_body(i, _):
        t_c = i - 2

        @pl.when(i < NITEM)
        def _():
            @pl.when(i >= NBUF)
            def _():
                wait_out(i - NBUF)

            start_item(i)

        wait_item(t_c)
        add_item(t_c)
        start_out(t_c)
        return 0

    lax.fori_loop(2, NITEM + 2, pipe_body, 0)

    for t in range(NITEM - NBUF, NITEM):
        wait_out(t)


def kernel(x, emb_table, pos_table):
    return _emb_lookup(x.astype(jnp.int32), emb_table, pos_table)
